# speedup vs baseline: 1.5468x; 1.5468x over previous
import functools

import jax
import jax.numpy as jnp
from jax import lax
from jax.experimental import pallas as pl
from jax.experimental.pallas import tpu as pltpu
from jax.experimental.pallas import tpu_sc as plsc

VOCAB = 50304
DIM = 768
BATCH = 2
SEQ = 2048
N_TOK = BATCH * SEQ
NC = 2
NS = 16
NW = NC * NS
B_PER_W = N_TOK // NW
N_TABLES = 6

_mesh = plsc.VectorSubcoreMesh(core_axis_name="c", subcore_axis_name="s")


@functools.partial(
    pl.kernel,
    out_type=[jax.ShapeDtypeStruct((N_TOK, DIM), jnp.float32)] * N_TABLES,
    mesh=_mesh,
    scratch_types=[
        pltpu.VMEM((B_PER_W,), jnp.int32),
        pltpu.VMEM((B_PER_W, DIM), jnp.float32),
        pltpu.SemaphoreType.DMA,
    ],
)
def _gather6(idx_hbm, t0, t1, t2, t3, t4, t5,
             o0, o1, o2, o3, o4, o5, idx_v, rows_v, sem):
    wid = lax.axis_index("s") * NC + lax.axis_index("c")
    base = wid * B_PER_W
    pltpu.sync_copy(idx_hbm.at[pl.ds(base, B_PER_W)], idx_v)
    for t, o in ((t0, o0), (t1, o1), (t2, o2), (t3, o3), (t4, o4), (t5, o5)):
        pltpu.async_copy(t.at[idx_v], rows_v, sem).wait()
        pltpu.sync_copy(rows_v, o.at[pl.ds(base, B_PER_W)])


def kernel(inputs, W0, W1, W2, W3, W4, W5):
    idx = inputs.reshape(N_TOK).astype(jnp.int32)
    outs = _gather6(idx, W0, W1, W2, W3, W4, W5)
    ve = [o.reshape(BATCH, SEQ, DIM) for o in outs]
    return tuple(ve + ve[::-1])


# speedup vs baseline: 1.5763x; 1.0191x over previous
import functools

import jax
import jax.numpy as jnp
from jax import lax
from jax.experimental import pallas as pl
from jax.experimental.pallas import tpu as pltpu
from jax.experimental.pallas import tpu_sc as plsc

VOCAB = 50304
DIM = 768
BATCH = 2
SEQ = 2048
N_TOK = BATCH * SEQ
NC = 2
NS = 16
NW = NC * NS
B_PER_W = N_TOK // NW
N_TABLES = 6

_mesh = plsc.VectorSubcoreMesh(core_axis_name="c", subcore_axis_name="s")


CH = 64
NCHUNK = B_PER_W // CH
NBUF = 2

@functools.partial(
    pl.kernel,
    out_type=[jax.ShapeDtypeStruct((N_TOK, DIM), jnp.float32)] * N_TABLES,
    mesh=_mesh,
    scratch_types=[
        pltpu.VMEM((B_PER_W,), jnp.int32),
    ] + [pltpu.VMEM((CH, DIM), jnp.float32)] * NBUF
      + [pltpu.SemaphoreType.DMA] * (2 * NBUF),
)
def _gather6(idx_hbm, t0, t1, t2, t3, t4, t5,
             o0, o1, o2, o3, o4, o5, idx_v,
             buf0, buf1, gs0, gs1, os0, os1):
    wid = lax.axis_index("s") * NC + lax.axis_index("c")
    base = wid * B_PER_W
    pltpu.sync_copy(idx_hbm.at[pl.ds(base, B_PER_W)], idx_v)
    tables = (t0, t1, t2, t3, t4, t5)
    outs = (o0, o1, o2, o3, o4, o5)
    bufs = (buf0, buf1)
    gsems = (gs0, gs1)
    osems = (os0, os1)
    steps = [(t, c) for t in range(N_TABLES) for c in range(NCHUNK)]
    nsteps = len(steps)

    def start_gather(s):
        t, c = steps[s]
        b = s % NBUF
        return pltpu.async_copy(
            tables[t].at[idx_v.at[pl.ds(c * CH, CH)]], bufs[b], gsems[b])

    def start_out(s):
        t, c = steps[s]
        b = s % NBUF
        return pltpu.async_copy(
            bufs[b], outs[t].at[pl.ds(base + c * CH, CH)], osems[b])

    g_desc = {0: start_gather(0)}
    o_desc = {}
    for s in range(nsteps):
        if s + 1 < nsteps:
            if s + 1 - NBUF >= 0:
                o_desc[s + 1 - NBUF].wait()
            g_desc[s + 1] = start_gather(s + 1)
        g_desc[s].wait()
        o_desc[s] = start_out(s)
    for s in range(nsteps - NBUF, nsteps):
        if s >= 0:
            o_desc[s].wait()


def kernel(inputs, W0, W1, W2, W3, W4, W5):
    idx = inputs.reshape(N_TOK).astype(jnp.int32)
    outs = _gather6(idx, W0, W1, W2, W3, W4, W5)
    ve = [o.reshape(BATCH, SEQ, DIM) for o in outs]
    return tuple(ve + ve[::-1])


# speedup vs baseline: 2.0554x; 1.3039x over previous
import functools

import jax
import jax.numpy as jnp
from jax import lax
from jax.experimental import pallas as pl
from jax.experimental.pallas import tpu as pltpu
from jax.experimental.pallas import tpu_sc as plsc

VOCAB = 50304
DIM = 768
BATCH = 2
SEQ = 2048
N_TOK = BATCH * SEQ
NC = 2
NS = 16
NW = NC * NS
B_PER_W = N_TOK // NW
N_TABLES = 6

_mesh = plsc.VectorSubcoreMesh(core_axis_name="c", subcore_axis_name="s")


CH = 64
NCHUNK = B_PER_W // CH
NBUF = 2

@functools.partial(
    pl.kernel,
    out_type=[jax.ShapeDtypeStruct((N_TOK, DIM), jnp.float32)] * (2 * N_TABLES),
    mesh=_mesh,
    scratch_types=[
        pltpu.VMEM((B_PER_W,), jnp.int32),
    ] + [pltpu.VMEM((CH, DIM), jnp.float32)] * NBUF
      + [pltpu.SemaphoreType.DMA] * (2 * NBUF),
)
def _gather6(idx_hbm, t0, t1, t2, t3, t4, t5,
             o0, o1, o2, o3, o4, o5, o6, o7, o8, o9, o10, o11, idx_v,
             buf0, buf1, gs0, gs1, os0, os1):
    wid = lax.axis_index("s") * NC + lax.axis_index("c")
    base = wid * B_PER_W
    pltpu.sync_copy(idx_hbm.at[pl.ds(base, B_PER_W)], idx_v)
    tables = (t0, t1, t2, t3, t4, t5)
    outs = (o0, o1, o2, o3, o4, o5, o6, o7, o8, o9, o10, o11)
    bufs = (buf0, buf1)
    gsems = (gs0, gs1)
    osems = (os0, os1)
    steps = [(t, c) for t in range(N_TABLES) for c in range(NCHUNK)]
    nsteps = len(steps)

    def start_gather(s):
        t, c = steps[s]
        b = s % NBUF
        return pltpu.async_copy(
            tables[t].at[idx_v.at[pl.ds(c * CH, CH)]], bufs[b], gsems[b])

    def start_out(s):
        t, c = steps[s]
        b = s % NBUF
        d1 = pltpu.async_copy(
            bufs[b], outs[t].at[pl.ds(base + c * CH, CH)], osems[b])
        d2 = pltpu.async_copy(
            bufs[b], outs[11 - t].at[pl.ds(base + c * CH, CH)], osems[b])
        return (d1, d2)

    g_desc = {0: start_gather(0)}
    o_desc = {}
    for s in range(nsteps):
        if s + 1 < nsteps:
            if s + 1 - NBUF >= 0:
                for d in o_desc[s + 1 - NBUF]:
                    d.wait()
            g_desc[s + 1] = start_gather(s + 1)
        g_desc[s].wait()
        o_desc[s] = start_out(s)
    for s in range(nsteps - NBUF, nsteps):
        if s >= 0:
            for d in o_desc[s]:
                d.wait()


def kernel(inputs, W0, W1, W2, W3, W4, W5):
    idx = inputs.reshape(N_TOK).astype(jnp.int32)
    outs = _gather6(idx, W0, W1, W2, W3, W4, W5)
    return tuple(o.reshape(BATCH, SEQ, DIM) for o in outs)
